# baseline (speedup 1.0000x reference)
import functools

import jax
import jax.numpy as jnp
from jax import lax
from jax.experimental import pallas as pl
from jax.experimental.pallas import tpu as pltpu
from jax.experimental.pallas import tpu_sc as plsc

E = 320000
DIM = 128
TBL = 3000
NUM_CORES = 2
NUM_SUBCORES = 16
NW = NUM_CORES * NUM_SUBCORES
B_PER_W = E // NW
CHUNK = 400
NCH = B_PER_W // CHUNK
SP_N = 320
HB_N = CHUNK - SP_N
LANES = 16


def _body(src_hbm, dst_hbm, table_hbm, out_hbm,
          src0, src1, dst0, dst1, idxv, rows0, rows1, table_sp,
          isem0, isem1, gsem0, gsem1, osem0, osem1, hsem0, hsem1):
    src_b = (src0, src1)
    dst_b = (dst0, dst1)
    rows_b = (rows0, rows1)
    isem = (isem0, isem1)
    gsem = (gsem0, gsem1)
    osem = (osem0, osem1)
    hsem = (hsem0, hsem1)

    wid = lax.axis_index("s") * NUM_CORES + lax.axis_index("c")
    base = wid * B_PER_W

    def fire_in(i, b):
        row0 = base + i * CHUNK
        pltpu.async_copy(src_hbm.at[pl.ds(row0, CHUNK)], src_b[b], isem[b])
        pltpu.async_copy(dst_hbm.at[pl.ds(row0, CHUNK)], dst_b[b], isem[b])

    def wait_in(i, b):
        row0 = base + i * CHUNK
        pltpu.make_async_copy(src_hbm.at[pl.ds(row0, CHUNK)], src_b[b], isem[b]).wait()
        pltpu.make_async_copy(dst_hbm.at[pl.ds(row0, CHUNK)], dst_b[b], isem[b]).wait()

    def compute(b):
        def f(j, c):
            x = src_b[b][pl.ds(j * LANES, LANES)]
            y = dst_b[b][pl.ds(j * LANES, LANES)]
            d = jnp.abs(x - y) - 1
            idxv[pl.ds(j * LANES, LANES)] = x * y + ((d * d) >> 2)
            return c

        lax.fori_loop(0, CHUNK // LANES, f, 0)

    def gather(b):
        g0 = pltpu.async_copy(
            table_sp.at[idxv.at[pl.ds(0, SP_N)]],
            rows_b[b].at[pl.ds(0, SP_N)],
            gsem[b],
        )
        g1 = pltpu.async_copy(
            table_hbm.at[idxv.at[pl.ds(SP_N, HB_N)]],
            rows_b[b].at[pl.ds(SP_N, HB_N)],
            hsem[b],
        )
        g0.wait()
        g1.wait()

    def fire_write(i, b):
        pltpu.async_copy(
            rows_b[b], out_hbm.at[pl.ds(base + i * CHUNK, CHUNK)], osem[b]
        )

    def wait_write(i, b):
        pltpu.make_async_copy(
            rows_b[b], out_hbm.at[pl.ds(base + i * CHUNK, CHUNK)], osem[b]
        ).wait()

    fire_in(0, 0)
    fire_in(1, 1)

    @pl.when(lax.axis_index("s") == 0)
    def _():
        pltpu.sync_copy(table_hbm, table_sp)

    plsc.subcore_barrier()

    wait_in(0, 0)
    compute(0)
    fire_in(2, 0)
    gather(0)
    fire_write(0, 0)

    wait_in(1, 1)
    compute(1)
    fire_in(3, 1)
    gather(1)
    fire_write(1, 1)

    def pair(it, c):
        a = 2 + 2 * it
        wait_in(a, 0)
        compute(0)
        fire_in(a + 2, 0)
        wait_write(a - 2, 0)
        gather(0)
        fire_write(a, 0)

        wait_in(a + 1, 1)
        compute(1)
        fire_in(a + 3, 1)
        wait_write(a - 1, 1)
        gather(1)
        fire_write(a + 1, 1)
        return c

    lax.fori_loop(0, (NCH - 5) // 2, pair, 0)

    wait_in(NCH - 3, 0)
    compute(0)
    fire_in(NCH - 1, 0)
    wait_write(NCH - 5, 0)
    gather(0)
    fire_write(NCH - 3, 0)

    wait_in(NCH - 2, 1)
    compute(1)
    wait_write(NCH - 4, 1)
    gather(1)
    fire_write(NCH - 2, 1)

    wait_in(NCH - 1, 0)
    compute(0)
    wait_write(NCH - 3, 0)
    gather(0)
    fire_write(NCH - 1, 0)

    wait_write(NCH - 2, 1)
    wait_write(NCH - 1, 0)


@jax.jit
def _run(src, dst, table):
    mesh = plsc.VectorSubcoreMesh(core_axis_name="c", subcore_axis_name="s")
    f = functools.partial(
        pl.kernel,
        mesh=mesh,
        out_type=jax.ShapeDtypeStruct((E, DIM), jnp.float32),
        scratch_types=[
            pltpu.VMEM((CHUNK,), jnp.int32),
            pltpu.VMEM((CHUNK,), jnp.int32),
            pltpu.VMEM((CHUNK,), jnp.int32),
            pltpu.VMEM((CHUNK,), jnp.int32),
            pltpu.VMEM((CHUNK,), jnp.int32),
            pltpu.VMEM((CHUNK, DIM), jnp.float32),
            pltpu.VMEM((CHUNK, DIM), jnp.float32),
            pltpu.VMEM_SHARED((TBL, DIM), jnp.float32),
            pltpu.SemaphoreType.DMA,
            pltpu.SemaphoreType.DMA,
            pltpu.SemaphoreType.DMA,
            pltpu.SemaphoreType.DMA,
            pltpu.SemaphoreType.DMA,
            pltpu.SemaphoreType.DMA,
            pltpu.SemaphoreType.DMA,
            pltpu.SemaphoreType.DMA,
        ],
    )(_body)
    return f(src, dst, table)


def kernel(src_node_type, dst_node_type, embedding):
    src = src_node_type.astype(jnp.int32)
    dst = dst_node_type.astype(jnp.int32)
    table = embedding.astype(jnp.float32)
    return _run(src, dst, table)


# speedup vs baseline: 1.1363x; 1.1363x over previous
import functools

import jax
import jax.numpy as jnp
from jax import lax
from jax.experimental import pallas as pl
from jax.experimental.pallas import tpu as pltpu
from jax.experimental.pallas import tpu_sc as plsc

E = 320000
DIM = 128
TBL = 3000
NUM_CORES = 2
NUM_SUBCORES = 16
NW = NUM_CORES * NUM_SUBCORES
B_PER_W = E // NW
CHUNK = 400
NCH = B_PER_W // CHUNK
SP_N = 360
HB_N = CHUNK - SP_N
LANES = 16


def _body(src_hbm, dst_hbm, table_hbm, out_hbm,
          src0, src1, dst0, dst1, idxv, rows0, rows1, table_sp,
          isem0, isem1, gsem0, gsem1, osem0, osem1, hsem0, hsem1):
    src_b = (src0, src1)
    dst_b = (dst0, dst1)
    rows_b = (rows0, rows1)
    isem = (isem0, isem1)
    gsem = (gsem0, gsem1)
    osem = (osem0, osem1)
    hsem = (hsem0, hsem1)

    wid = lax.axis_index("s") * NUM_CORES + lax.axis_index("c")
    base = wid * B_PER_W

    def fire_in(i, b):
        row0 = base + i * CHUNK
        pltpu.async_copy(src_hbm.at[pl.ds(row0, CHUNK)], src_b[b], isem[b])
        pltpu.async_copy(dst_hbm.at[pl.ds(row0, CHUNK)], dst_b[b], isem[b])

    def wait_in(i, b):
        row0 = base + i * CHUNK
        pltpu.make_async_copy(src_hbm.at[pl.ds(row0, CHUNK)], src_b[b], isem[b]).wait()
        pltpu.make_async_copy(dst_hbm.at[pl.ds(row0, CHUNK)], dst_b[b], isem[b]).wait()

    def compute(b):
        def f(j, c):
            x = src_b[b][pl.ds(j * LANES, LANES)]
            y = dst_b[b][pl.ds(j * LANES, LANES)]
            d = jnp.abs(x - y) - 1
            idxv[pl.ds(j * LANES, LANES)] = x * y + ((d * d) >> 2)
            return c

        lax.fori_loop(0, CHUNK // LANES, f, 0)

    def gather(b):
        g0 = pltpu.async_copy(
            table_sp.at[idxv.at[pl.ds(0, SP_N)]],
            rows_b[b].at[pl.ds(0, SP_N)],
            gsem[b],
        )
        g1 = pltpu.async_copy(
            table_hbm.at[idxv.at[pl.ds(SP_N, HB_N)]],
            rows_b[b].at[pl.ds(SP_N, HB_N)],
            hsem[b],
        )
        g0.wait()
        g1.wait()

    def fire_write(i, b):
        pltpu.async_copy(
            rows_b[b], out_hbm.at[pl.ds(base + i * CHUNK, CHUNK)], osem[b]
        )

    def wait_write(i, b):
        pltpu.make_async_copy(
            rows_b[b], out_hbm.at[pl.ds(base + i * CHUNK, CHUNK)], osem[b]
        ).wait()

    fire_in(0, 0)
    fire_in(1, 1)

    @pl.when(lax.axis_index("s") == 0)
    def _():
        pltpu.sync_copy(table_hbm, table_sp)

    plsc.subcore_barrier()

    wait_in(0, 0)
    compute(0)
    fire_in(2, 0)
    gather(0)
    fire_write(0, 0)

    wait_in(1, 1)
    compute(1)
    fire_in(3, 1)
    gather(1)
    fire_write(1, 1)

    def pair(it, c):
        a = 2 + 2 * it
        wait_in(a, 0)
        compute(0)
        fire_in(a + 2, 0)
        wait_write(a - 2, 0)
        gather(0)
        fire_write(a, 0)

        wait_in(a + 1, 1)
        compute(1)
        fire_in(a + 3, 1)
        wait_write(a - 1, 1)
        gather(1)
        fire_write(a + 1, 1)
        return c

    lax.fori_loop(0, (NCH - 5) // 2, pair, 0)

    wait_in(NCH - 3, 0)
    compute(0)
    fire_in(NCH - 1, 0)
    wait_write(NCH - 5, 0)
    gather(0)
    fire_write(NCH - 3, 0)

    wait_in(NCH - 2, 1)
    compute(1)
    wait_write(NCH - 4, 1)
    gather(1)
    fire_write(NCH - 2, 1)

    wait_in(NCH - 1, 0)
    compute(0)
    wait_write(NCH - 3, 0)
    gather(0)
    fire_write(NCH - 1, 0)

    wait_write(NCH - 2, 1)
    wait_write(NCH - 1, 0)


@jax.jit
def _run(src, dst, table):
    mesh = plsc.VectorSubcoreMesh(core_axis_name="c", subcore_axis_name="s")
    f = functools.partial(
        pl.kernel,
        mesh=mesh,
        out_type=jax.ShapeDtypeStruct((E, DIM), jnp.float32),
        scratch_types=[
            pltpu.VMEM((CHUNK,), jnp.int32),
            pltpu.VMEM((CHUNK,), jnp.int32),
            pltpu.VMEM((CHUNK,), jnp.int32),
            pltpu.VMEM((CHUNK,), jnp.int32),
            pltpu.VMEM((CHUNK,), jnp.int32),
            pltpu.VMEM((CHUNK, DIM), jnp.float32),
            pltpu.VMEM((CHUNK, DIM), jnp.float32),
            pltpu.VMEM_SHARED((TBL, DIM), jnp.float32),
            pltpu.SemaphoreType.DMA,
            pltpu.SemaphoreType.DMA,
            pltpu.SemaphoreType.DMA,
            pltpu.SemaphoreType.DMA,
            pltpu.SemaphoreType.DMA,
            pltpu.SemaphoreType.DMA,
            pltpu.SemaphoreType.DMA,
            pltpu.SemaphoreType.DMA,
        ],
    )(_body)
    return f(src, dst, table)


def kernel(src_node_type, dst_node_type, embedding):
    src = src_node_type.astype(jnp.int32)
    dst = dst_node_type.astype(jnp.int32)
    table = embedding.astype(jnp.float32)
    return _run(src, dst, table)


# speedup vs baseline: 1.2252x; 1.0782x over previous
import functools

import jax
import jax.numpy as jnp
from jax import lax
from jax.experimental import pallas as pl
from jax.experimental.pallas import tpu as pltpu
from jax.experimental.pallas import tpu_sc as plsc

E = 320000
DIM = 128
TBL = 3000
NUM_CORES = 2
NUM_SUBCORES = 16
NW = NUM_CORES * NUM_SUBCORES
B_PER_W = E // NW
CHUNK = 400
NCH = B_PER_W // CHUNK
LANES = 16


def _body(src_hbm, dst_hbm, table_hbm, out_hbm,
          src0, src1, dst0, dst1, idx0, idx1, rows0, rows1, table_sp,
          isem0, isem1, gsem0, gsem1, osem0, osem1):
    src_b = (src0, src1)
    dst_b = (dst0, dst1)
    idx_b = (idx0, idx1)
    rows_b = (rows0, rows1)
    isem = (isem0, isem1)
    gsem = (gsem0, gsem1)
    osem = (osem0, osem1)

    wid = lax.axis_index("s") * NUM_CORES + lax.axis_index("c")
    base = wid * B_PER_W

    def fire_in(i, b):
        row0 = base + i * CHUNK
        pltpu.async_copy(src_hbm.at[pl.ds(row0, CHUNK)], src_b[b], isem[b])
        pltpu.async_copy(dst_hbm.at[pl.ds(row0, CHUNK)], dst_b[b], isem[b])

    def wait_in(i, b):
        row0 = base + i * CHUNK
        pltpu.make_async_copy(src_hbm.at[pl.ds(row0, CHUNK)], src_b[b], isem[b]).wait()
        pltpu.make_async_copy(dst_hbm.at[pl.ds(row0, CHUNK)], dst_b[b], isem[b]).wait()

    def compute(b):
        def f(j, c):
            x = src_b[b][pl.ds(j * LANES, LANES)]
            y = dst_b[b][pl.ds(j * LANES, LANES)]
            d = jnp.abs(x - y) - 1
            idx_b[b][pl.ds(j * LANES, LANES)] = x * y + ((d * d) >> 2)
            return c

        lax.fori_loop(0, CHUNK // LANES, f, 0)

    def fire_gather(b):
        return pltpu.async_copy(table_sp.at[idx_b[b]], rows_b[b], gsem[b])

    def fire_write(i, b):
        pltpu.async_copy(
            rows_b[b], out_hbm.at[pl.ds(base + i * CHUNK, CHUNK)], osem[b]
        )

    def wait_write(i, b):
        pltpu.make_async_copy(
            rows_b[b], out_hbm.at[pl.ds(base + i * CHUNK, CHUNK)], osem[b]
        ).wait()

    fire_in(0, 0)
    fire_in(1, 1)

    @pl.when(lax.axis_index("s") == 0)
    def _():
        pltpu.sync_copy(table_hbm, table_sp)

    plsc.subcore_barrier()

    wait_in(0, 0)
    compute(0)
    fire_in(2, 0)
    g = fire_gather(0)
    wait_in(1, 1)
    compute(1)
    fire_in(3, 1)
    g.wait()
    fire_write(0, 0)

    g = fire_gather(1)
    wait_in(2, 0)
    compute(0)
    fire_in(4, 0)
    g.wait()
    fire_write(1, 1)

    def pair(it, c):
        a = 2 + 2 * it
        wait_write(a - 2, 0)
        g0 = fire_gather(0)
        wait_in(a + 1, 1)
        compute(1)
        fire_in(a + 3, 1)
        g0.wait()
        fire_write(a, 0)

        wait_write(a - 1, 1)
        g1 = fire_gather(1)
        wait_in(a + 2, 0)
        compute(0)
        fire_in(a + 4, 0)
        g1.wait()
        fire_write(a + 1, 1)
        return c

    lax.fori_loop(0, (NCH - 5) // 2, pair, 0)

    wait_write(NCH - 5, 0)
    g = fire_gather(0)
    wait_in(NCH - 2, 1)
    compute(1)
    g.wait()
    fire_write(NCH - 3, 0)

    wait_write(NCH - 4, 1)
    g = fire_gather(1)
    wait_in(NCH - 1, 0)
    compute(0)
    g.wait()
    fire_write(NCH - 2, 1)

    wait_write(NCH - 3, 0)
    g = fire_gather(0)
    g.wait()
    fire_write(NCH - 1, 0)

    wait_write(NCH - 2, 1)
    wait_write(NCH - 1, 0)


@jax.jit
def _run(src, dst, table):
    mesh = plsc.VectorSubcoreMesh(core_axis_name="c", subcore_axis_name="s")
    f = functools.partial(
        pl.kernel,
        mesh=mesh,
        out_type=jax.ShapeDtypeStruct((E, DIM), jnp.float32),
        scratch_types=[
            pltpu.VMEM((CHUNK,), jnp.int32),
            pltpu.VMEM((CHUNK,), jnp.int32),
            pltpu.VMEM((CHUNK,), jnp.int32),
            pltpu.VMEM((CHUNK,), jnp.int32),
            pltpu.VMEM((CHUNK,), jnp.int32),
            pltpu.VMEM((CHUNK,), jnp.int32),
            pltpu.VMEM((CHUNK, DIM), jnp.float32),
            pltpu.VMEM((CHUNK, DIM), jnp.float32),
            pltpu.VMEM_SHARED((TBL, DIM), jnp.float32),
            pltpu.SemaphoreType.DMA,
            pltpu.SemaphoreType.DMA,
            pltpu.SemaphoreType.DMA,
            pltpu.SemaphoreType.DMA,
            pltpu.SemaphoreType.DMA,
            pltpu.SemaphoreType.DMA,
        ],
    )(_body)
    return f(src, dst, table)


def kernel(src_node_type, dst_node_type, embedding):
    src = src_node_type.astype(jnp.int32)
    dst = dst_node_type.astype(jnp.int32)
    table = embedding.astype(jnp.float32)
    return _run(src, dst, table)
